# Initial kernel scaffold; baseline (speedup 1.0000x reference)
import jax
import jax.numpy as jnp
from jax.experimental import pallas as pl


def kernel(x, mp_edge_index, pred_edge_index, pred_edge_attr, W1l, b1l, W1r, W2l, b2l, W2r, Wm1, bm1, Wm2, bm2):
    raise NotImplementedError("write your pallas kernel here")


# speedup vs baseline: 3.1849x; 3.1849x over previous
import functools

import jax
import jax.numpy as jnp
from jax import lax
from jax.experimental import pallas as pl
from jax.experimental.pallas import tpu as pltpu
from jax.experimental.pallas import tpu_sc as plsc

_N = 10000
_E = 320000
_D = 128
_H = 64

_NC = 2
_NS = 16
_NW = _NC * _NS
_CH = 128
_KCH = 79
_EPT = _CH * _KCH
_NP = _NW * _EPT
_W1 = 80
_ACC = 10240
_RPS = _ACC // _NS

_MESH = plsc.VectorSubcoreMesh(
    core_axis_name="c", subcore_axis_name="s", num_cores=_NC, num_subcores=_NS
)


def _seg_sum(table, src, dst, zeros, width):
    @functools.partial(
        pl.kernel,
        out_type=jax.ShapeDtypeStruct((_NC, _ACC, width), jnp.float32),
        mesh=_MESH,
        scratch_types=[
            pltpu.VMEM((_CH,), jnp.int32),
            pltpu.VMEM((_CH,), jnp.int32),
            pltpu.VMEM((_CH, width), jnp.float32),
            pltpu.VMEM_SHARED((_ACC, width), jnp.float32),
            pltpu.SemaphoreType.DMA,
        ],
        compiler_params=pltpu.CompilerParams(use_tc_tiling_on_sc=False),
    )
    def k(table_h, src_h, dst_h, zeros_h, out_h, sidx, didx, rows, acc, sem):
        c = lax.axis_index("c")
        s = lax.axis_index("s")
        wid = s * _NC + c
        pltpu.sync_copy(zeros_h.at[pl.ds(s * _RPS, _RPS)],
                        acc.at[pl.ds(s * _RPS, _RPS)])
        plsc.subcore_barrier()
        base = wid * _EPT
        def body(i, carry):
            off = pl.multiple_of(base + i * _CH, 8)
            pltpu.sync_copy(src_h.at[pl.ds(off, _CH)], sidx)
            pltpu.sync_copy(dst_h.at[pl.ds(off, _CH)], didx)
            pltpu.async_copy(table_h.at[sidx], rows, sem).wait()
            pltpu.sync_copy(rows, acc.at[didx], add=True)
            return carry
        lax.fori_loop(0, _KCH, body, 0)
        plsc.subcore_barrier()
        pltpu.sync_copy(acc.at[pl.ds(s * _RPS, _RPS)],
                        out_h.at[c].at[pl.ds(s * _RPS, _RPS)])

    return k(table, src, dst, zeros)


def _edge_gather(p, q, src, dst):
    @functools.partial(
        pl.kernel,
        out_type=(
            jax.ShapeDtypeStruct((_NP, _H), jnp.float32),
            jax.ShapeDtypeStruct((_NP, _H), jnp.float32),
        ),
        mesh=_MESH,
        scratch_types=[
            pltpu.VMEM((_CH,), jnp.int32),
            pltpu.VMEM((_CH,), jnp.int32),
            pltpu.VMEM((_CH, _H), jnp.float32),
            pltpu.VMEM((_CH, _H), jnp.float32),
            pltpu.SemaphoreType.DMA,
            pltpu.SemaphoreType.DMA,
        ],
        compiler_params=pltpu.CompilerParams(use_tc_tiling_on_sc=False),
    )
    def k(p_h, q_h, src_h, dst_h, gs_h, gq_h, sidx, didx, prows, qrows, s1, s2):
        c = lax.axis_index("c")
        s = lax.axis_index("s")
        wid = s * _NC + c
        base = wid * _EPT
        def body(i, carry):
            off = pl.multiple_of(base + i * _CH, 8)
            pltpu.sync_copy(src_h.at[pl.ds(off, _CH)], sidx)
            pltpu.sync_copy(dst_h.at[pl.ds(off, _CH)], didx)
            cp1 = pltpu.async_copy(p_h.at[sidx], prows, s1)
            cp2 = pltpu.async_copy(q_h.at[didx], qrows, s2)
            cp1.wait()
            cp2.wait()
            pltpu.sync_copy(prows, gs_h.at[pl.ds(off, _CH)])
            pltpu.sync_copy(qrows, gq_h.at[pl.ds(off, _CH)])
            return carry
        lax.fori_loop(0, _KCH, body, 0)

    return k(p, q, src, dst)


def _tc_pre(x, w1lt, w1rt, b1l):
    blk = 1000

    def body(x_r, wl_r, wr_r, b_r, y1p_r, z1_r):
        xb = x_r[...]
        y = jnp.dot(xb, wl_r[...], preferred_element_type=jnp.float32)
        y1p_r[...] = jnp.concatenate(
            [y, jnp.ones((blk, 1), jnp.float32),
             jnp.zeros((blk, _W1 - _H - 1), jnp.float32)], axis=1)
        z1_r[...] = jnp.dot(xb, wr_r[...], preferred_element_type=jnp.float32) + b_r[...]

    return pl.pallas_call(
        body,
        grid=(_N // blk,),
        in_specs=[
            pl.BlockSpec((blk, _D), lambda i: (i, 0)),
            pl.BlockSpec((_D, _H), lambda i: (0, 0)),
            pl.BlockSpec((_D, _H), lambda i: (0, 0)),
            pl.BlockSpec((1, _H), lambda i: (0, 0)),
        ],
        out_specs=[
            pl.BlockSpec((blk, _W1), lambda i: (i, 0)),
            pl.BlockSpec((blk, _H), lambda i: (i, 0)),
        ],
        out_shape=[
            jax.ShapeDtypeStruct((_N, _W1), jnp.float32),
            jax.ShapeDtypeStruct((_N, _H), jnp.float32),
        ],
    )(x, w1lt, w1rt, b1l)


def _tc_mid(sa, sb, z1, w2lt, w2rt, b2l):
    blk = 1000

    def body(sa_r, sb_r, z_r, wl_r, wr_r, b_r, y2_r, z2_r, rc_r):
        ss = sa_r[...] + sb_r[...]
        rcnt = 1.0 / jnp.maximum(ss[:, _H:_H + 1], 1.0)
        h1 = jnp.maximum(ss[:, :_H] * rcnt + z_r[...], 0.0)
        y2_r[...] = jnp.dot(h1, wl_r[...], preferred_element_type=jnp.float32)
        z2_r[...] = jnp.dot(h1, wr_r[...], preferred_element_type=jnp.float32) + b_r[...]
        rc_r[...] = rcnt

    return pl.pallas_call(
        body,
        grid=(_N // blk,),
        in_specs=[
            pl.BlockSpec((blk, _W1), lambda i: (i, 0)),
            pl.BlockSpec((blk, _W1), lambda i: (i, 0)),
            pl.BlockSpec((blk, _H), lambda i: (i, 0)),
            pl.BlockSpec((_H, _H), lambda i: (0, 0)),
            pl.BlockSpec((_H, _H), lambda i: (0, 0)),
            pl.BlockSpec((1, _H), lambda i: (0, 0)),
        ],
        out_specs=[
            pl.BlockSpec((blk, _H), lambda i: (i, 0)),
            pl.BlockSpec((blk, _H), lambda i: (i, 0)),
            pl.BlockSpec((blk, 1), lambda i: (i, 0)),
        ],
        out_shape=[
            jax.ShapeDtypeStruct((_N, _H), jnp.float32),
            jax.ShapeDtypeStruct((_N, _H), jnp.float32),
            jax.ShapeDtypeStruct((_N, 1), jnp.float32),
        ],
    )(sa, sb, z1, w2lt, w2rt, b2l)


def _tc_post(sa, sb, z2, rcnt, at, bt):
    blk = 1000

    def body(sa_r, sb_r, z_r, rc_r, a_r, b_r, p_r, q_r):
        h2 = jnp.maximum((sa_r[...] + sb_r[...]) * rc_r[...] + z_r[...], 0.0)
        p_r[...] = jnp.dot(h2, a_r[...], preferred_element_type=jnp.float32)
        q_r[...] = jnp.dot(h2, b_r[...], preferred_element_type=jnp.float32)

    return pl.pallas_call(
        body,
        grid=(_N // blk,),
        in_specs=[
            pl.BlockSpec((blk, _H), lambda i: (i, 0)),
            pl.BlockSpec((blk, _H), lambda i: (i, 0)),
            pl.BlockSpec((blk, _H), lambda i: (i, 0)),
            pl.BlockSpec((blk, 1), lambda i: (i, 0)),
            pl.BlockSpec((_H, _H), lambda i: (0, 0)),
            pl.BlockSpec((_H, _H), lambda i: (0, 0)),
        ],
        out_specs=[
            pl.BlockSpec((blk, _H), lambda i: (i, 0)),
            pl.BlockSpec((blk, _H), lambda i: (i, 0)),
        ],
        out_shape=[
            jax.ShapeDtypeStruct((_N, _H), jnp.float32),
            jax.ShapeDtypeStruct((_N, _H), jnp.float32),
        ],
    )(sa, sb, z2, rcnt, at, bt)


def _tc_r(attr, ct, bm1):
    blk = 2048

    def body(a_r, c_r, b_r, r_r):
        r_r[...] = jnp.dot(a_r[...], c_r[...], preferred_element_type=jnp.float32) + b_r[...]

    return pl.pallas_call(
        body,
        grid=(_NP // blk,),
        in_specs=[
            pl.BlockSpec((blk, 16), lambda i: (i, 0)),
            pl.BlockSpec((16, _H), lambda i: (0, 0)),
            pl.BlockSpec((1, _H), lambda i: (0, 0)),
        ],
        out_specs=pl.BlockSpec((blk, _H), lambda i: (i, 0)),
        out_shape=jax.ShapeDtypeStruct((_NP, _H), jnp.float32),
    )(attr, ct, bm1)


def _tc_final(gs, gq, r, wm2c, bm2):
    blk = 2048

    def body(gs_r, gq_r, r_r, w_r, b_r, o_r):
        hid = jnp.maximum(gs_r[...] + gq_r[...] + r_r[...], 0.0)
        o_r[...] = jnp.dot(hid, w_r[...], preferred_element_type=jnp.float32) + b_r[...]

    return pl.pallas_call(
        body,
        grid=(_NP // blk,),
        in_specs=[
            pl.BlockSpec((blk, _H), lambda i: (i, 0)),
            pl.BlockSpec((blk, _H), lambda i: (i, 0)),
            pl.BlockSpec((blk, _H), lambda i: (i, 0)),
            pl.BlockSpec((_H, 1), lambda i: (0, 0)),
            pl.BlockSpec((1, 1), lambda i: (0, 0)),
        ],
        out_specs=pl.BlockSpec((blk, 1), lambda i: (i, 0)),
        out_shape=jax.ShapeDtypeStruct((_NP, 1), jnp.float32),
    )(gs, gq, r, wm2c, bm2)


def kernel(x, mp_edge_index, pred_edge_index, pred_edge_attr,
           W1l, b1l, W1r, W2l, b2l, W2r, Wm1, bm1, Wm2, bm2):
    f32 = jnp.float32
    pad_e = _NP - _E
    mp_src = jnp.pad(mp_edge_index[0], (0, pad_e))
    mp_dst = jnp.pad(mp_edge_index[1], (0, pad_e), constant_values=_N)
    pr_src = jnp.pad(pred_edge_index[0], (0, pad_e))
    pr_dst = jnp.pad(pred_edge_index[1], (0, pad_e))
    attr_p = jnp.pad(pred_edge_attr, ((0, pad_e), (0, 0)))
    zeros80 = jnp.zeros((_ACC, _W1), f32)
    zeros64 = jnp.zeros((_ACC, _H), f32)

    y1p, z1 = _tc_pre(x, W1l.T, W1r.T, b1l.reshape(1, _H))
    s1 = _seg_sum(y1p, mp_src, mp_dst, zeros80, _W1)
    y2, z2, rcnt = _tc_mid(s1[0, :_N], s1[1, :_N], z1,
                           W2l.T, W2r.T, b2l.reshape(1, _H))
    s2 = _seg_sum(y2, mp_src, mp_dst, zeros64, _H)
    p, q = _tc_post(s2[0, :_N], s2[1, :_N], z2, rcnt,
                    Wm1[:, :_H].T, Wm1[:, _H:2 * _H].T)
    r = _tc_r(attr_p, Wm1[:, 2 * _H:].T, bm1.reshape(1, _H))
    gs, gq = _edge_gather(p, q, pr_src, pr_dst)
    o = _tc_final(gs, gq, r, Wm2.T, bm2.reshape(1, 1))
    return o[:_E, 0]


# speedup vs baseline: 3.3895x; 1.0642x over previous
import functools

import jax
import jax.numpy as jnp
from jax import lax
from jax.experimental import pallas as pl
from jax.experimental.pallas import tpu as pltpu
from jax.experimental.pallas import tpu_sc as plsc

_N = 10000
_E = 320000
_D = 128
_H = 64

_NC = 2
_NS = 16
_NW = _NC * _NS
_CH = 128
_KCH = 80
_EPT = _CH * _KCH
_NP = _NW * _EPT
_W1 = 80
_ACC = 10240
_RPS = _ACC // _NS

_MESH = plsc.VectorSubcoreMesh(
    core_axis_name="c", subcore_axis_name="s", num_cores=_NC, num_subcores=_NS
)


def _seg_sum(table, src, dst, zeros, width):
    @functools.partial(
        pl.kernel,
        out_type=jax.ShapeDtypeStruct((_NC, _ACC, width), jnp.float32),
        mesh=_MESH,
        scratch_types=[
            pltpu.VMEM((_CH,), jnp.int32),
            pltpu.VMEM((_CH,), jnp.int32),
            pltpu.VMEM((_CH,), jnp.int32),
            pltpu.VMEM((_CH,), jnp.int32),
            pltpu.VMEM((_CH, width), jnp.float32),
            pltpu.VMEM((_CH, width), jnp.float32),
            pltpu.VMEM_SHARED((_ACC, width), jnp.float32),
            pltpu.SemaphoreType.DMA,
            pltpu.SemaphoreType.DMA,
        ],
        compiler_params=pltpu.CompilerParams(use_tc_tiling_on_sc=False),
    )
    def k(table_h, src_h, dst_h, zeros_h, out_h,
          sidx0, sidx1, didx0, didx1, rows0, rows1, acc, sem0, sem1):
        c = lax.axis_index("c")
        s = lax.axis_index("s")
        wid = s * _NC + c
        pltpu.sync_copy(zeros_h.at[pl.ds(s * _RPS, _RPS)],
                        acc.at[pl.ds(s * _RPS, _RPS)])
        plsc.subcore_barrier()
        base = wid * _EPT
        sidx = (sidx0, sidx1)
        didx = (didx0, didx1)
        rows = (rows0, rows1)
        sem = (sem0, sem1)

        def stage_a(cc, j):
            off = pl.multiple_of(base + cc * _CH, 8)
            pltpu.sync_copy(src_h.at[pl.ds(off, _CH)], sidx[j])
            pltpu.sync_copy(dst_h.at[pl.ds(off, _CH)], didx[j])
            pltpu.async_copy(table_h.at[sidx[j]], rows[j], sem[j])

        def stage_b(j):
            pltpu.make_async_copy(table_h.at[sidx[j]], rows[j], sem[j]).wait()
            pltpu.sync_copy(rows[j], acc.at[didx[j]], add=True)

        stage_a(0, 0)
        stage_a(1, 1)

        def body(g, carry):
            stage_b(0)
            stage_a(2 * g + 2, 0)
            stage_b(1)
            stage_a(2 * g + 3, 1)
            return carry

        lax.fori_loop(0, _KCH // 2 - 1, body, 0)
        stage_b(0)
        stage_b(1)
        plsc.subcore_barrier()
        pltpu.sync_copy(acc.at[pl.ds(s * _RPS, _RPS)],
                        out_h.at[c].at[pl.ds(s * _RPS, _RPS)])

    return k(table, src, dst, zeros)


def _edge_gather(p, q, src, dst):
    nb = 4

    @functools.partial(
        pl.kernel,
        out_type=(
            jax.ShapeDtypeStruct((_NP, _H), jnp.float32),
            jax.ShapeDtypeStruct((_NP, _H), jnp.float32),
        ),
        mesh=_MESH,
        scratch_types=(
            [pltpu.VMEM((_CH,), jnp.int32) for _ in range(2 * nb)]
            + [pltpu.VMEM((_CH, _H), jnp.float32) for _ in range(2 * nb)]
            + [pltpu.SemaphoreType.DMA for _ in range(2 * nb)]
        ),
        compiler_params=pltpu.CompilerParams(use_tc_tiling_on_sc=False),
    )
    def k(p_h, q_h, src_h, dst_h, gs_h, gq_h, *scr):
        sidx = scr[0:nb]
        didx = scr[nb:2 * nb]
        prows = scr[2 * nb:3 * nb]
        qrows = scr[3 * nb:4 * nb]
        gsem = scr[4 * nb:5 * nb]
        wsem = scr[5 * nb:6 * nb]
        c = lax.axis_index("c")
        s = lax.axis_index("s")
        wid = s * _NC + c
        base = wid * _EPT

        def stage_a(cc, j):
            off = pl.multiple_of(base + cc * _CH, 8)
            pltpu.sync_copy(src_h.at[pl.ds(off, _CH)], sidx[j])
            pltpu.sync_copy(dst_h.at[pl.ds(off, _CH)], didx[j])
            pltpu.async_copy(p_h.at[sidx[j]], prows[j], gsem[j])
            pltpu.async_copy(q_h.at[didx[j]], qrows[j], gsem[j])

        def stage_b(cc, j):
            off = pl.multiple_of(base + cc * _CH, 8)
            pltpu.make_async_copy(p_h.at[sidx[j]], prows[j], gsem[j]).wait()
            pltpu.make_async_copy(q_h.at[didx[j]], qrows[j], gsem[j]).wait()
            pltpu.async_copy(prows[j], gs_h.at[pl.ds(off, _CH)], wsem[j])
            pltpu.async_copy(qrows[j], gq_h.at[pl.ds(off, _CH)], wsem[j])

        def wait_w(cc, j):
            off = pl.multiple_of(base + cc * _CH, 8)
            pltpu.make_async_copy(prows[j], gs_h.at[pl.ds(off, _CH)], wsem[j]).wait()
            pltpu.make_async_copy(qrows[j], gq_h.at[pl.ds(off, _CH)], wsem[j]).wait()

        for j in range(nb):
            stage_a(j, j)
            if j > 0:
                stage_b(j - 1, j - 1)

        def body(g, carry):
            for j in range(nb):
                cc = g * nb + j
                wait_w(cc - nb, j)
                stage_a(cc, j)
                stage_b(cc - 1, (j - 1) % nb)
            return carry

        lax.fori_loop(1, _KCH // nb, body, 0)
        stage_b(_KCH - 1, nb - 1)
        for j in range(nb):
            wait_w(_KCH - nb + j, j)

    return k(p, q, src, dst)


def _tc_pre(x, w1lt, w1rt, b1l):
    blk = 1000

    def body(x_r, wl_r, wr_r, b_r, y1p_r, z1_r):
        xb = x_r[...]
        y = jnp.dot(xb, wl_r[...], preferred_element_type=jnp.float32)
        y1p_r[...] = jnp.concatenate(
            [y, jnp.ones((blk, 1), jnp.float32),
             jnp.zeros((blk, _W1 - _H - 1), jnp.float32)], axis=1)
        z1_r[...] = jnp.dot(xb, wr_r[...], preferred_element_type=jnp.float32) + b_r[...]

    return pl.pallas_call(
        body,
        grid=(_N // blk,),
        in_specs=[
            pl.BlockSpec((blk, _D), lambda i: (i, 0)),
            pl.BlockSpec((_D, _H), lambda i: (0, 0)),
            pl.BlockSpec((_D, _H), lambda i: (0, 0)),
            pl.BlockSpec((1, _H), lambda i: (0, 0)),
        ],
        out_specs=[
            pl.BlockSpec((blk, _W1), lambda i: (i, 0)),
            pl.BlockSpec((blk, _H), lambda i: (i, 0)),
        ],
        out_shape=[
            jax.ShapeDtypeStruct((_N, _W1), jnp.float32),
            jax.ShapeDtypeStruct((_N, _H), jnp.float32),
        ],
    )(x, w1lt, w1rt, b1l)


def _tc_mid(sa, sb, z1, w2lt, w2rt, b2l):
    blk = 1000

    def body(sa_r, sb_r, z_r, wl_r, wr_r, b_r, y2_r, z2_r, rc_r):
        ss = sa_r[...] + sb_r[...]
        rcnt = 1.0 / jnp.maximum(ss[:, _H:_H + 1], 1.0)
        h1 = jnp.maximum(ss[:, :_H] * rcnt + z_r[...], 0.0)
        y2_r[...] = jnp.dot(h1, wl_r[...], preferred_element_type=jnp.float32)
        z2_r[...] = jnp.dot(h1, wr_r[...], preferred_element_type=jnp.float32) + b_r[...]
        rc_r[...] = rcnt

    return pl.pallas_call(
        body,
        grid=(_N // blk,),
        in_specs=[
            pl.BlockSpec((blk, _W1), lambda i: (i, 0)),
            pl.BlockSpec((blk, _W1), lambda i: (i, 0)),
            pl.BlockSpec((blk, _H), lambda i: (i, 0)),
            pl.BlockSpec((_H, _H), lambda i: (0, 0)),
            pl.BlockSpec((_H, _H), lambda i: (0, 0)),
            pl.BlockSpec((1, _H), lambda i: (0, 0)),
        ],
        out_specs=[
            pl.BlockSpec((blk, _H), lambda i: (i, 0)),
            pl.BlockSpec((blk, _H), lambda i: (i, 0)),
            pl.BlockSpec((blk, 1), lambda i: (i, 0)),
        ],
        out_shape=[
            jax.ShapeDtypeStruct((_N, _H), jnp.float32),
            jax.ShapeDtypeStruct((_N, _H), jnp.float32),
            jax.ShapeDtypeStruct((_N, 1), jnp.float32),
        ],
    )(sa, sb, z1, w2lt, w2rt, b2l)


def _tc_post(sa, sb, z2, rcnt, at, bt):
    blk = 1000

    def body(sa_r, sb_r, z_r, rc_r, a_r, b_r, p_r, q_r):
        h2 = jnp.maximum((sa_r[...] + sb_r[...]) * rc_r[...] + z_r[...], 0.0)
        p_r[...] = jnp.dot(h2, a_r[...], preferred_element_type=jnp.float32)
        q_r[...] = jnp.dot(h2, b_r[...], preferred_element_type=jnp.float32)

    return pl.pallas_call(
        body,
        grid=(_N // blk,),
        in_specs=[
            pl.BlockSpec((blk, _H), lambda i: (i, 0)),
            pl.BlockSpec((blk, _H), lambda i: (i, 0)),
            pl.BlockSpec((blk, _H), lambda i: (i, 0)),
            pl.BlockSpec((blk, 1), lambda i: (i, 0)),
            pl.BlockSpec((_H, _H), lambda i: (0, 0)),
            pl.BlockSpec((_H, _H), lambda i: (0, 0)),
        ],
        out_specs=[
            pl.BlockSpec((blk, _H), lambda i: (i, 0)),
            pl.BlockSpec((blk, _H), lambda i: (i, 0)),
        ],
        out_shape=[
            jax.ShapeDtypeStruct((_N, _H), jnp.float32),
            jax.ShapeDtypeStruct((_N, _H), jnp.float32),
        ],
    )(sa, sb, z2, rcnt, at, bt)


def _tc_final(gs, gq, attr, ct, bm1, wm2c, bm2):
    blk = 2048

    def body(gs_r, gq_r, a_r, c_r, b1_r, w_r, b2_r, o_r):
        r = jnp.dot(a_r[...], c_r[...], preferred_element_type=jnp.float32) + b1_r[...]
        hid = jnp.maximum(gs_r[...] + gq_r[...] + r, 0.0)
        o_r[...] = jnp.dot(hid, w_r[...], preferred_element_type=jnp.float32) + b2_r[...]

    return pl.pallas_call(
        body,
        grid=(_NP // blk,),
        in_specs=[
            pl.BlockSpec((blk, _H), lambda i: (i, 0)),
            pl.BlockSpec((blk, _H), lambda i: (i, 0)),
            pl.BlockSpec((blk, 16), lambda i: (i, 0)),
            pl.BlockSpec((16, _H), lambda i: (0, 0)),
            pl.BlockSpec((1, _H), lambda i: (0, 0)),
            pl.BlockSpec((_H, 1), lambda i: (0, 0)),
            pl.BlockSpec((1, 1), lambda i: (0, 0)),
        ],
        out_specs=pl.BlockSpec((blk, 1), lambda i: (i, 0)),
        out_shape=jax.ShapeDtypeStruct((_NP, 1), jnp.float32),
    )(gs, gq, attr, ct, bm1, wm2c, bm2)


def kernel(x, mp_edge_index, pred_edge_index, pred_edge_attr,
           W1l, b1l, W1r, W2l, b2l, W2r, Wm1, bm1, Wm2, bm2):
    f32 = jnp.float32
    pad_e = _NP - _E
    mp_src = jnp.pad(mp_edge_index[0], (0, pad_e))
    mp_dst = jnp.pad(mp_edge_index[1], (0, pad_e), constant_values=_N)
    pr_src = jnp.pad(pred_edge_index[0], (0, pad_e))
    pr_dst = jnp.pad(pred_edge_index[1], (0, pad_e))
    attr_p = jnp.pad(pred_edge_attr, ((0, pad_e), (0, 0)))
    zeros80 = jnp.zeros((_ACC, _W1), f32)
    zeros64 = jnp.zeros((_ACC, _H), f32)

    y1p, z1 = _tc_pre(x, W1l.T, W1r.T, b1l.reshape(1, _H))
    s1 = _seg_sum(y1p, mp_src, mp_dst, zeros80, _W1)
    y2, z2, rcnt = _tc_mid(s1[0, :_N], s1[1, :_N], z1,
                           W2l.T, W2r.T, b2l.reshape(1, _H))
    s2 = _seg_sum(y2, mp_src, mp_dst, zeros64, _H)
    p, q = _tc_post(s2[0, :_N], s2[1, :_N], z2, rcnt,
                    Wm1[:, :_H].T, Wm1[:, _H:2 * _H].T)
    gs, gq = _edge_gather(p, q, pr_src, pr_dst)
    o = _tc_final(gs, gq, attr_p, Wm1[:, 2 * _H:].T,
                  bm1.reshape(1, _H), Wm2.T, bm2.reshape(1, 1))
    return o[:_E, 0]


# speedup vs baseline: 3.8345x; 1.1313x over previous
import functools

import jax
import jax.numpy as jnp
from jax import lax
from jax.experimental import pallas as pl
from jax.experimental.pallas import tpu as pltpu
from jax.experimental.pallas import tpu_sc as plsc

_N = 10000
_E = 320000
_D = 128
_H = 64

_NC = 2
_NS = 16
_NW = _NC * _NS
_CH = 128
_KCH = 80
_EPT = _CH * _KCH
_NP = _NW * _EPT
_W1 = 96
_ACC = 10240
_RPS = _ACC // _NS

_MESH = plsc.VectorSubcoreMesh(
    core_axis_name="c", subcore_axis_name="s", num_cores=_NC, num_subcores=_NS
)


def _seg_sum(table, src, dst, zeros, width):
    @functools.partial(
        pl.kernel,
        out_type=jax.ShapeDtypeStruct((_NC, _ACC, width), jnp.bfloat16),
        mesh=_MESH,
        scratch_types=[
            pltpu.VMEM((_CH,), jnp.int32),
            pltpu.VMEM((_CH,), jnp.int32),
            pltpu.VMEM((_CH,), jnp.int32),
            pltpu.VMEM((_CH,), jnp.int32),
            pltpu.VMEM((_CH, width), jnp.bfloat16),
            pltpu.VMEM((_CH, width), jnp.bfloat16),
            pltpu.VMEM_SHARED((_ACC, width), jnp.bfloat16),
            pltpu.SemaphoreType.DMA,
            pltpu.SemaphoreType.DMA,
        ],
        compiler_params=pltpu.CompilerParams(use_tc_tiling_on_sc=False),
    )
    def k(table_h, src_h, dst_h, zeros_h, out_h,
          sidx0, sidx1, didx0, didx1, rows0, rows1, acc, sem0, sem1):
        c = lax.axis_index("c")
        s = lax.axis_index("s")
        wid = s * _NC + c
        pltpu.sync_copy(zeros_h.at[pl.ds(s * _RPS, _RPS)],
                        acc.at[pl.ds(s * _RPS, _RPS)])
        plsc.subcore_barrier()
        base = wid * _EPT
        sidx = (sidx0, sidx1)
        didx = (didx0, didx1)
        rows = (rows0, rows1)
        sem = (sem0, sem1)

        def stage_a(cc, j):
            off = pl.multiple_of(base + cc * _CH, 8)
            pltpu.sync_copy(src_h.at[pl.ds(off, _CH)], sidx[j])
            pltpu.sync_copy(dst_h.at[pl.ds(off, _CH)], didx[j])
            pltpu.async_copy(table_h.at[sidx[j]], rows[j], sem[j])

        def stage_b(j):
            pltpu.make_async_copy(table_h.at[sidx[j]], rows[j], sem[j]).wait()
            pltpu.sync_copy(rows[j], acc.at[didx[j]], add=True)

        stage_a(0, 0)
        stage_a(1, 1)

        def body(g, carry):
            stage_b(0)
            stage_a(2 * g + 2, 0)
            stage_b(1)
            stage_a(2 * g + 3, 1)
            return carry

        lax.fori_loop(0, _KCH // 2 - 1, body, 0)
        stage_b(0)
        stage_b(1)
        plsc.subcore_barrier()
        pltpu.sync_copy(acc.at[pl.ds(s * _RPS, _RPS)],
                        out_h.at[c].at[pl.ds(s * _RPS, _RPS)])

    return k(table, src, dst, zeros)


def _edge_gather(p, q, src, dst):
    nb = 4

    @functools.partial(
        pl.kernel,
        out_type=(
            jax.ShapeDtypeStruct((_NP, _H), jnp.bfloat16),
            jax.ShapeDtypeStruct((_NP, _H), jnp.bfloat16),
        ),
        mesh=_MESH,
        scratch_types=(
            [pltpu.VMEM((_CH,), jnp.int32) for _ in range(2 * nb)]
            + [pltpu.VMEM((_CH, _H), jnp.bfloat16) for _ in range(2 * nb)]
            + [pltpu.SemaphoreType.DMA for _ in range(2 * nb)]
        ),
        compiler_params=pltpu.CompilerParams(use_tc_tiling_on_sc=False),
    )
    def k(p_h, q_h, src_h, dst_h, gs_h, gq_h, *scr):
        sidx = scr[0:nb]
        didx = scr[nb:2 * nb]
        prows = scr[2 * nb:3 * nb]
        qrows = scr[3 * nb:4 * nb]
        gsem = scr[4 * nb:5 * nb]
        wsem = scr[5 * nb:6 * nb]
        c = lax.axis_index("c")
        s = lax.axis_index("s")
        wid = s * _NC + c
        base = wid * _EPT

        def stage_a(cc, j):
            off = pl.multiple_of(base + cc * _CH, 8)
            pltpu.sync_copy(src_h.at[pl.ds(off, _CH)], sidx[j])
            pltpu.sync_copy(dst_h.at[pl.ds(off, _CH)], didx[j])
            pltpu.async_copy(p_h.at[sidx[j]], prows[j], gsem[j])
            pltpu.async_copy(q_h.at[didx[j]], qrows[j], gsem[j])

        def stage_b(cc, j):
            off = pl.multiple_of(base + cc * _CH, 8)
            pltpu.make_async_copy(p_h.at[sidx[j]], prows[j], gsem[j]).wait()
            pltpu.make_async_copy(q_h.at[didx[j]], qrows[j], gsem[j]).wait()
            pltpu.async_copy(prows[j], gs_h.at[pl.ds(off, _CH)], wsem[j])
            pltpu.async_copy(qrows[j], gq_h.at[pl.ds(off, _CH)], wsem[j])

        def wait_w(cc, j):
            off = pl.multiple_of(base + cc * _CH, 8)
            pltpu.make_async_copy(prows[j], gs_h.at[pl.ds(off, _CH)], wsem[j]).wait()
            pltpu.make_async_copy(qrows[j], gq_h.at[pl.ds(off, _CH)], wsem[j]).wait()

        for j in range(nb):
            stage_a(j, j)
            if j > 0:
                stage_b(j - 1, j - 1)

        def body(g, carry):
            for j in range(nb):
                cc = g * nb + j
                wait_w(cc - nb, j)
                stage_a(cc, j)
                stage_b(cc - 1, (j - 1) % nb)
            return carry

        lax.fori_loop(1, _KCH // nb, body, 0)
        stage_b(_KCH - 1, nb - 1)
        for j in range(nb):
            wait_w(_KCH - nb + j, j)

    return k(p, q, src, dst)


def _tc_pre(x, w1lt, w1rt, b1l):
    blk = 1000

    def body(x_r, wl_r, wr_r, b_r, y1p_r, z1_r):
        xb = x_r[...]
        y = jnp.dot(xb, wl_r[...], preferred_element_type=jnp.float32)
        y1p_r[...] = jnp.concatenate(
            [y, jnp.ones((blk, 1), jnp.float32),
             jnp.zeros((blk, _W1 - _H - 1), jnp.float32)], axis=1).astype(jnp.bfloat16)
        z1_r[...] = jnp.dot(xb, wr_r[...], preferred_element_type=jnp.float32) + b_r[...]

    return pl.pallas_call(
        body,
        grid=(_N // blk,),
        in_specs=[
            pl.BlockSpec((blk, _D), lambda i: (i, 0)),
            pl.BlockSpec((_D, _H), lambda i: (0, 0)),
            pl.BlockSpec((_D, _H), lambda i: (0, 0)),
            pl.BlockSpec((1, _H), lambda i: (0, 0)),
        ],
        out_specs=[
            pl.BlockSpec((blk, _W1), lambda i: (i, 0)),
            pl.BlockSpec((blk, _H), lambda i: (i, 0)),
        ],
        out_shape=[
            jax.ShapeDtypeStruct((_N, _W1), jnp.bfloat16),
            jax.ShapeDtypeStruct((_N, _H), jnp.float32),
        ],
    )(x, w1lt, w1rt, b1l)


def _tc_mid(sa, sb, z1, w2lt, w2rt, b2l):
    blk = 1000

    def body(sa_r, sb_r, z_r, wl_r, wr_r, b_r, y2_r, z2_r, rc_r):
        ss = sa_r[...].astype(jnp.float32) + sb_r[...].astype(jnp.float32)
        rcnt = 1.0 / jnp.maximum(ss[:, _H:_H + 1], 1.0)
        h1 = jnp.maximum(ss[:, :_H] * rcnt + z_r[...], 0.0)
        y2_r[...] = jnp.dot(h1, wl_r[...], preferred_element_type=jnp.float32).astype(jnp.bfloat16)
        z2_r[...] = jnp.dot(h1, wr_r[...], preferred_element_type=jnp.float32) + b_r[...]
        rc_r[...] = rcnt

    return pl.pallas_call(
        body,
        grid=(_N // blk,),
        in_specs=[
            pl.BlockSpec((blk, _W1), lambda i: (i, 0)),
            pl.BlockSpec((blk, _W1), lambda i: (i, 0)),
            pl.BlockSpec((blk, _H), lambda i: (i, 0)),
            pl.BlockSpec((_H, _H), lambda i: (0, 0)),
            pl.BlockSpec((_H, _H), lambda i: (0, 0)),
            pl.BlockSpec((1, _H), lambda i: (0, 0)),
        ],
        out_specs=[
            pl.BlockSpec((blk, _H), lambda i: (i, 0)),
            pl.BlockSpec((blk, _H), lambda i: (i, 0)),
            pl.BlockSpec((blk, 1), lambda i: (i, 0)),
        ],
        out_shape=[
            jax.ShapeDtypeStruct((_N, _H), jnp.bfloat16),
            jax.ShapeDtypeStruct((_N, _H), jnp.float32),
            jax.ShapeDtypeStruct((_N, 1), jnp.float32),
        ],
    )(sa, sb, z1, w2lt, w2rt, b2l)


def _tc_post(sa, sb, z2, rcnt, at, bt):
    blk = 1000

    def body(sa_r, sb_r, z_r, rc_r, a_r, b_r, p_r, q_r):
        h2 = jnp.maximum(
            (sa_r[...].astype(jnp.float32) + sb_r[...].astype(jnp.float32))
            * rc_r[...] + z_r[...], 0.0)
        p_r[...] = jnp.dot(h2, a_r[...], preferred_element_type=jnp.float32).astype(jnp.bfloat16)
        q_r[...] = jnp.dot(h2, b_r[...], preferred_element_type=jnp.float32).astype(jnp.bfloat16)

    return pl.pallas_call(
        body,
        grid=(_N // blk,),
        in_specs=[
            pl.BlockSpec((blk, _H), lambda i: (i, 0)),
            pl.BlockSpec((blk, _H), lambda i: (i, 0)),
            pl.BlockSpec((blk, _H), lambda i: (i, 0)),
            pl.BlockSpec((blk, 1), lambda i: (i, 0)),
            pl.BlockSpec((_H, _H), lambda i: (0, 0)),
            pl.BlockSpec((_H, _H), lambda i: (0, 0)),
        ],
        out_specs=[
            pl.BlockSpec((blk, _H), lambda i: (i, 0)),
            pl.BlockSpec((blk, _H), lambda i: (i, 0)),
        ],
        out_shape=[
            jax.ShapeDtypeStruct((_N, _H), jnp.bfloat16),
            jax.ShapeDtypeStruct((_N, _H), jnp.bfloat16),
        ],
    )(sa, sb, z2, rcnt, at, bt)


def _tc_final(gs, gq, attr, ct, bm1, wm2c, bm2):
    blk = 2048

    def body(gs_r, gq_r, a_r, c_r, b1_r, w_r, b2_r, o_r):
        r = jnp.dot(a_r[...], c_r[...], preferred_element_type=jnp.float32) + b1_r[...]
        hid = jnp.maximum(gs_r[...].astype(jnp.float32)
                          + gq_r[...].astype(jnp.float32) + r, 0.0)
        o_r[...] = jnp.dot(hid, w_r[...], preferred_element_type=jnp.float32) + b2_r[...]

    return pl.pallas_call(
        body,
        grid=(_NP // blk,),
        in_specs=[
            pl.BlockSpec((blk, _H), lambda i: (i, 0)),
            pl.BlockSpec((blk, _H), lambda i: (i, 0)),
            pl.BlockSpec((blk, 16), lambda i: (i, 0)),
            pl.BlockSpec((16, _H), lambda i: (0, 0)),
            pl.BlockSpec((1, _H), lambda i: (0, 0)),
            pl.BlockSpec((_H, 1), lambda i: (0, 0)),
            pl.BlockSpec((1, 1), lambda i: (0, 0)),
        ],
        out_specs=pl.BlockSpec((blk, 1), lambda i: (i, 0)),
        out_shape=jax.ShapeDtypeStruct((_NP, 1), jnp.float32),
    )(gs, gq, attr, ct, bm1, wm2c, bm2)


def kernel(x, mp_edge_index, pred_edge_index, pred_edge_attr,
           W1l, b1l, W1r, W2l, b2l, W2r, Wm1, bm1, Wm2, bm2):
    f32 = jnp.float32
    pad_e = _NP - _E
    mp_src = jnp.pad(mp_edge_index[0], (0, pad_e))
    mp_dst = jnp.pad(mp_edge_index[1], (0, pad_e), constant_values=_N)
    pr_src = jnp.pad(pred_edge_index[0], (0, pad_e))
    pr_dst = jnp.pad(pred_edge_index[1], (0, pad_e))
    attr_p = jnp.pad(pred_edge_attr, ((0, pad_e), (0, 0)))
    zeros96 = jnp.zeros((_ACC, _W1), jnp.bfloat16)
    zeros64 = jnp.zeros((_ACC, _H), jnp.bfloat16)

    y1p, z1 = _tc_pre(x, W1l.T, W1r.T, b1l.reshape(1, _H))
    s1 = _seg_sum(y1p, mp_src, mp_dst, zeros96, _W1)
    y2, z2, rcnt = _tc_mid(s1[0, :_N], s1[1, :_N], z1,
                           W2l.T, W2r.T, b2l.reshape(1, _H))
    s2 = _seg_sum(y2, mp_src, mp_dst, zeros64, _H)
    p, q = _tc_post(s2[0, :_N], s2[1, :_N], z2, rcnt,
                    Wm1[:, :_H].T, Wm1[:, _H:2 * _H].T)
    gs, gq = _edge_gather(p, q, pr_src, pr_dst)
    o = _tc_final(gs, gq, attr_p, Wm1[:, 2 * _H:].T,
                  bm1.reshape(1, _H), Wm2.T, bm2.reshape(1, 1))
    return o[:_E, 0]


# speedup vs baseline: 3.9219x; 1.0228x over previous
import functools

import jax
import jax.numpy as jnp
from jax import lax
from jax.experimental import pallas as pl
from jax.experimental.pallas import tpu as pltpu
from jax.experimental.pallas import tpu_sc as plsc

_N = 10000
_E = 320000
_D = 128
_H = 64

_NC = 2
_NS = 16
_NW = _NC * _NS
_CH = 128
_KCH = 80
_EPT = _CH * _KCH
_NP = _NW * _EPT
_W1 = 96
_ACC = 10240
_RPS = _ACC // _NS

_MESH = plsc.VectorSubcoreMesh(
    core_axis_name="c", subcore_axis_name="s", num_cores=_NC, num_subcores=_NS
)


def _seg_sum(table, src3, dst3, zeros, width):
    nb = 4

    @functools.partial(
        pl.kernel,
        out_type=jax.ShapeDtypeStruct((_NC, _ACC, width), jnp.bfloat16),
        mesh=_MESH,
        scratch_types=(
            [pltpu.VMEM((_KCH, _CH), jnp.int32) for _ in range(2)]
            + [pltpu.VMEM((_CH, width), jnp.bfloat16) for _ in range(nb)]
            + [pltpu.VMEM_SHARED((_ACC, width), jnp.bfloat16)]
            + [pltpu.SemaphoreType.DMA for _ in range(2 * nb)]
        ),
        compiler_params=pltpu.CompilerParams(use_tc_tiling_on_sc=False),
    )
    def k(table_h, src3_h, dst3_h, zeros_h, out_h, *scr):
        sidx_all, didx_all = scr[0], scr[1]
        rows = scr[2:2 + nb]
        acc = scr[2 + nb]
        gsem = scr[3 + nb:3 + 2 * nb]
        asem = scr[3 + 2 * nb:3 + 3 * nb]
        c = lax.axis_index("c")
        s = lax.axis_index("s")
        wid = s * _NC + c
        pltpu.sync_copy(src3_h.at[wid], sidx_all)
        pltpu.sync_copy(dst3_h.at[wid], didx_all)
        pltpu.sync_copy(zeros_h.at[pl.ds(s * _RPS, _RPS)],
                        acc.at[pl.ds(s * _RPS, _RPS)])
        plsc.subcore_barrier()

        def ga(cc, j):
            pltpu.make_async_copy(table_h.at[sidx_all.at[cc]], rows[j],
                                  gsem[j]).start()

        def wg_sa(cc, j):
            pltpu.make_async_copy(table_h.at[sidx_all.at[cc]], rows[j],
                                  gsem[j]).wait()
            pltpu.make_async_copy(rows[j], acc.at[didx_all.at[cc]],
                                  asem[j]).start(add=True)

        def wa(cc, j):
            pltpu.make_async_copy(rows[j], acc.at[didx_all.at[cc]],
                                  asem[j]).wait()

        for j in range(nb):
            ga(j, j)
            if j > 0:
                wg_sa(j - 1, j - 1)

        def body(g, carry):
            for j in range(nb):
                cc = g * nb + j
                wa(cc - nb, j)
                ga(cc, j)
                wg_sa(cc - 1, (j - 1) % nb)
            return carry

        lax.fori_loop(1, _KCH // nb, body, 0)
        wg_sa(_KCH - 1, nb - 1)
        for j in range(nb):
            wa(_KCH - nb + j, j)
        plsc.subcore_barrier()
        pltpu.sync_copy(acc.at[pl.ds(s * _RPS, _RPS)],
                        out_h.at[c].at[pl.ds(s * _RPS, _RPS)])

    return k(table, src3, dst3, zeros)


def _edge_gather(p, q, src3, dst3):
    nb = 4

    @functools.partial(
        pl.kernel,
        out_type=(
            jax.ShapeDtypeStruct((_NP, _H), jnp.bfloat16),
            jax.ShapeDtypeStruct((_NP, _H), jnp.bfloat16),
        ),
        mesh=_MESH,
        scratch_types=(
            [pltpu.VMEM((_KCH, _CH), jnp.int32) for _ in range(2)]
            + [pltpu.VMEM((_CH, _H), jnp.bfloat16) for _ in range(2 * nb)]
            + [pltpu.SemaphoreType.DMA for _ in range(2 * nb)]
        ),
        compiler_params=pltpu.CompilerParams(use_tc_tiling_on_sc=False),
    )
    def k(p_h, q_h, src3_h, dst3_h, gs_h, gq_h, *scr):
        sidx_all, didx_all = scr[0], scr[1]
        prows = scr[2:2 + nb]
        qrows = scr[2 + nb:2 + 2 * nb]
        gsem = scr[2 + 2 * nb:2 + 3 * nb]
        wsem = scr[2 + 3 * nb:2 + 4 * nb]
        c = lax.axis_index("c")
        s = lax.axis_index("s")
        wid = s * _NC + c
        base = wid * _EPT
        pltpu.sync_copy(src3_h.at[wid], sidx_all)
        pltpu.sync_copy(dst3_h.at[wid], didx_all)

        def stage_a(cc, j):
            pltpu.make_async_copy(p_h.at[sidx_all.at[cc]], prows[j],
                                  gsem[j]).start()
            pltpu.make_async_copy(q_h.at[didx_all.at[cc]], qrows[j],
                                  gsem[j]).start()

        def stage_b(cc, j):
            off = pl.multiple_of(base + cc * _CH, 8)
            pltpu.make_async_copy(p_h.at[sidx_all.at[cc]], prows[j],
                                  gsem[j]).wait()
            pltpu.make_async_copy(q_h.at[didx_all.at[cc]], qrows[j],
                                  gsem[j]).wait()
            pltpu.async_copy(prows[j], gs_h.at[pl.ds(off, _CH)], wsem[j])
            pltpu.async_copy(qrows[j], gq_h.at[pl.ds(off, _CH)], wsem[j])

        def wait_w(cc, j):
            off = pl.multiple_of(base + cc * _CH, 8)
            pltpu.make_async_copy(prows[j], gs_h.at[pl.ds(off, _CH)], wsem[j]).wait()
            pltpu.make_async_copy(qrows[j], gq_h.at[pl.ds(off, _CH)], wsem[j]).wait()

        for j in range(nb):
            stage_a(j, j)
            if j > 0:
                stage_b(j - 1, j - 1)

        def body(g, carry):
            for j in range(nb):
                cc = g * nb + j
                wait_w(cc - nb, j)
                stage_a(cc, j)
                stage_b(cc - 1, (j - 1) % nb)
            return carry

        lax.fori_loop(1, _KCH // nb, body, 0)
        stage_b(_KCH - 1, nb - 1)
        for j in range(nb):
            wait_w(_KCH - nb + j, j)

    return k(p, q, src3, dst3)


def _tc_pre(x, w1lt, w1rt, b1l):
    blk = 1000

    def body(x_r, wl_r, wr_r, b_r, y1p_r, z1_r):
        xb = x_r[...]
        y = jnp.dot(xb, wl_r[...], preferred_element_type=jnp.float32)
        y1p_r[...] = jnp.concatenate(
            [y, jnp.ones((blk, 1), jnp.float32),
             jnp.zeros((blk, _W1 - _H - 1), jnp.float32)], axis=1).astype(jnp.bfloat16)
        z1_r[...] = jnp.dot(xb, wr_r[...], preferred_element_type=jnp.float32) + b_r[...]

    return pl.pallas_call(
        body,
        grid=(_N // blk,),
        in_specs=[
            pl.BlockSpec((blk, _D), lambda i: (i, 0)),
            pl.BlockSpec((_D, _H), lambda i: (0, 0)),
            pl.BlockSpec((_D, _H), lambda i: (0, 0)),
            pl.BlockSpec((1, _H), lambda i: (0, 0)),
        ],
        out_specs=[
            pl.BlockSpec((blk, _W1), lambda i: (i, 0)),
            pl.BlockSpec((blk, _H), lambda i: (i, 0)),
        ],
        out_shape=[
            jax.ShapeDtypeStruct((_N, _W1), jnp.bfloat16),
            jax.ShapeDtypeStruct((_N, _H), jnp.float32),
        ],
    )(x, w1lt, w1rt, b1l)


def _tc_mid(sa, sb, z1, w2lt, w2rt, b2l):
    blk = 1000

    def body(sa_r, sb_r, z_r, wl_r, wr_r, b_r, y2_r, z2_r, rc_r):
        ss = sa_r[...].astype(jnp.float32) + sb_r[...].astype(jnp.float32)
        rcnt = 1.0 / jnp.maximum(ss[:, _H:_H + 1], 1.0)
        h1 = jnp.maximum(ss[:, :_H] * rcnt + z_r[...], 0.0)
        y2_r[...] = jnp.dot(h1, wl_r[...], preferred_element_type=jnp.float32).astype(jnp.bfloat16)
        z2_r[...] = jnp.dot(h1, wr_r[...], preferred_element_type=jnp.float32) + b_r[...]
        rc_r[...] = rcnt

    return pl.pallas_call(
        body,
        grid=(_N // blk,),
        in_specs=[
            pl.BlockSpec((blk, _W1), lambda i: (i, 0)),
            pl.BlockSpec((blk, _W1), lambda i: (i, 0)),
            pl.BlockSpec((blk, _H), lambda i: (i, 0)),
            pl.BlockSpec((_H, _H), lambda i: (0, 0)),
            pl.BlockSpec((_H, _H), lambda i: (0, 0)),
            pl.BlockSpec((1, _H), lambda i: (0, 0)),
        ],
        out_specs=[
            pl.BlockSpec((blk, _H), lambda i: (i, 0)),
            pl.BlockSpec((blk, _H), lambda i: (i, 0)),
            pl.BlockSpec((blk, 1), lambda i: (i, 0)),
        ],
        out_shape=[
            jax.ShapeDtypeStruct((_N, _H), jnp.bfloat16),
            jax.ShapeDtypeStruct((_N, _H), jnp.float32),
            jax.ShapeDtypeStruct((_N, 1), jnp.float32),
        ],
    )(sa, sb, z1, w2lt, w2rt, b2l)


def _tc_post(sa, sb, z2, rcnt, at, bt):
    blk = 1000

    def body(sa_r, sb_r, z_r, rc_r, a_r, b_r, p_r, q_r):
        h2 = jnp.maximum(
            (sa_r[...].astype(jnp.float32) + sb_r[...].astype(jnp.float32))
            * rc_r[...] + z_r[...], 0.0)
        p_r[...] = jnp.dot(h2, a_r[...], preferred_element_type=jnp.float32).astype(jnp.bfloat16)
        q_r[...] = jnp.dot(h2, b_r[...], preferred_element_type=jnp.float32).astype(jnp.bfloat16)

    return pl.pallas_call(
        body,
        grid=(_N // blk,),
        in_specs=[
            pl.BlockSpec((blk, _H), lambda i: (i, 0)),
            pl.BlockSpec((blk, _H), lambda i: (i, 0)),
            pl.BlockSpec((blk, _H), lambda i: (i, 0)),
            pl.BlockSpec((blk, 1), lambda i: (i, 0)),
            pl.BlockSpec((_H, _H), lambda i: (0, 0)),
            pl.BlockSpec((_H, _H), lambda i: (0, 0)),
        ],
        out_specs=[
            pl.BlockSpec((blk, _H), lambda i: (i, 0)),
            pl.BlockSpec((blk, _H), lambda i: (i, 0)),
        ],
        out_shape=[
            jax.ShapeDtypeStruct((_N, _H), jnp.bfloat16),
            jax.ShapeDtypeStruct((_N, _H), jnp.bfloat16),
        ],
    )(sa, sb, z2, rcnt, at, bt)


def _tc_final(gs, gq, attr, ct, bm1, wm2c, bm2):
    blk = 2048

    def body(gs_r, gq_r, a_r, c_r, b1_r, w_r, b2_r, o_r):
        r = jnp.dot(a_r[...], c_r[...], preferred_element_type=jnp.float32) + b1_r[...]
        hid = jnp.maximum(gs_r[...].astype(jnp.float32)
                          + gq_r[...].astype(jnp.float32) + r, 0.0)
        o_r[...] = jnp.dot(hid, w_r[...], preferred_element_type=jnp.float32) + b2_r[...]

    return pl.pallas_call(
        body,
        grid=(_NP // blk,),
        in_specs=[
            pl.BlockSpec((blk, _H), lambda i: (i, 0)),
            pl.BlockSpec((blk, _H), lambda i: (i, 0)),
            pl.BlockSpec((blk, 16), lambda i: (i, 0)),
            pl.BlockSpec((16, _H), lambda i: (0, 0)),
            pl.BlockSpec((1, _H), lambda i: (0, 0)),
            pl.BlockSpec((_H, 1), lambda i: (0, 0)),
            pl.BlockSpec((1, 1), lambda i: (0, 0)),
        ],
        out_specs=pl.BlockSpec((blk, 1), lambda i: (i, 0)),
        out_shape=jax.ShapeDtypeStruct((_NP, 1), jnp.float32),
    )(gs, gq, attr, ct, bm1, wm2c, bm2)


def kernel(x, mp_edge_index, pred_edge_index, pred_edge_attr,
           W1l, b1l, W1r, W2l, b2l, W2r, Wm1, bm1, Wm2, bm2):
    f32 = jnp.float32
    pad_e = _NP - _E
    mp_src = jnp.pad(mp_edge_index[0], (0, pad_e)).reshape(_NW, _KCH, _CH)
    mp_dst = jnp.pad(mp_edge_index[1], (0, pad_e),
                     constant_values=_N).reshape(_NW, _KCH, _CH)
    pr_src = jnp.pad(pred_edge_index[0], (0, pad_e)).reshape(_NW, _KCH, _CH)
    pr_dst = jnp.pad(pred_edge_index[1], (0, pad_e)).reshape(_NW, _KCH, _CH)
    attr_p = jnp.pad(pred_edge_attr, ((0, pad_e), (0, 0)))
    zeros96 = jnp.zeros((_ACC, _W1), jnp.bfloat16)
    zeros64 = jnp.zeros((_ACC, _H), jnp.bfloat16)

    y1p, z1 = _tc_pre(x, W1l.T, W1r.T, b1l.reshape(1, _H))
    s1 = _seg_sum(y1p, mp_src, mp_dst, zeros96, _W1)
    y2, z2, rcnt = _tc_mid(s1[0, :_N], s1[1, :_N], z1,
                           W2l.T, W2r.T, b2l.reshape(1, _H))
    s2 = _seg_sum(y2, mp_src, mp_dst, zeros64, _H)
    p, q = _tc_post(s2[0, :_N], s2[1, :_N], z2, rcnt,
                    Wm1[:, :_H].T, Wm1[:, _H:2 * _H].T)
    gs, gq = _edge_gather(p, q, pr_src, pr_dst)
    o = _tc_final(gs, gq, attr_p, Wm1[:, 2 * _H:].T,
                  bm1.reshape(1, _H), Wm2.T, bm2.reshape(1, 1))
    return o[:_E, 0]


# speedup vs baseline: 5.1855x; 1.3222x over previous
import functools

import jax
import jax.numpy as jnp
from jax import lax
from jax.experimental import pallas as pl
from jax.experimental.pallas import tpu as pltpu
from jax.experimental.pallas import tpu_sc as plsc

_N = 10000
_E = 320000
_D = 128
_H = 64

_NC = 2
_NS = 16
_NW = _NC * _NS
_CH = 128
_KCH = 80
_EPT = _CH * _KCH
_NP = _NW * _EPT
_W1 = 96
_ACC = 10240
_RPS = _ACC // _NS

_MESH = plsc.VectorSubcoreMesh(
    core_axis_name="c", subcore_axis_name="s", num_cores=_NC, num_subcores=_NS
)


def _seg_sum(table, src3, dst3, zeros, width):
    nb = 4

    @functools.partial(
        pl.kernel,
        out_type=jax.ShapeDtypeStruct((_NC, _ACC, width), jnp.bfloat16),
        mesh=_MESH,
        scratch_types=(
            [pltpu.VMEM((_KCH, _CH), jnp.int32) for _ in range(2)]
            + [pltpu.VMEM((_CH, width), jnp.bfloat16) for _ in range(nb)]
            + [pltpu.VMEM_SHARED((_ACC, width), jnp.bfloat16)]
            + [pltpu.SemaphoreType.DMA for _ in range(2 * nb)]
        ),
        compiler_params=pltpu.CompilerParams(use_tc_tiling_on_sc=False,
                                             needs_layout_passes=False),
    )
    def k(table_h, src3_h, dst3_h, zeros_h, out_h, *scr):
        sidx_all, didx_all = scr[0], scr[1]
        rows = scr[2:2 + nb]
        acc = scr[2 + nb]
        gsem = scr[3 + nb:3 + 2 * nb]
        asem = scr[3 + 2 * nb:3 + 3 * nb]
        c = lax.axis_index("c")
        s = lax.axis_index("s")
        wid = s * _NC + c
        pltpu.sync_copy(src3_h.at[wid], sidx_all)
        pltpu.sync_copy(dst3_h.at[wid], didx_all)
        pltpu.sync_copy(zeros_h.at[pl.ds(s * _RPS, _RPS)],
                        acc.at[pl.ds(s * _RPS, _RPS)])
        plsc.subcore_barrier()

        def ga(cc, j):
            pltpu.make_async_copy(table_h.at[sidx_all.at[cc]], rows[j],
                                  gsem[j]).start()

        def wg_sa(cc, j):
            pltpu.make_async_copy(table_h.at[sidx_all.at[cc]], rows[j],
                                  gsem[j]).wait()
            pltpu.make_async_copy(rows[j], acc.at[didx_all.at[cc]],
                                  asem[j]).start(add=True)

        def wa(cc, j):
            pltpu.make_async_copy(rows[j], acc.at[didx_all.at[cc]],
                                  asem[j]).wait()

        for j in range(nb):
            ga(j, j)
            if j > 0:
                wg_sa(j - 1, j - 1)

        def body(g, carry):
            for j in range(nb):
                cc = g * nb + j
                wa(cc - nb, j)
                ga(cc, j)
                wg_sa(cc - 1, (j - 1) % nb)
            return carry

        lax.fori_loop(1, _KCH // nb, body, 0)
        wg_sa(_KCH - 1, nb - 1)
        for j in range(nb):
            wa(_KCH - nb + j, j)
        plsc.subcore_barrier()
        pltpu.sync_copy(acc.at[pl.ds(s * _RPS, _RPS)],
                        out_h.at[c].at[pl.ds(s * _RPS, _RPS)])

    return k(table, src3, dst3, zeros)


def _edge_mlp(p, q, r, src3, dst3, wm2_sc, bm2v):
    nb = 4

    @functools.partial(
        pl.kernel,
        out_type=jax.ShapeDtypeStruct((_NP // 16, 16), jnp.float32),
        mesh=_MESH,
        scratch_types=(
            [pltpu.VMEM((_KCH, _CH), jnp.int32) for _ in range(2)]
            + [pltpu.VMEM((_CH, _H), jnp.bfloat16) for _ in range(3 * nb)]
            + [pltpu.VMEM((_CH // 16, 16), jnp.float32) for _ in range(nb)]
            + [pltpu.VMEM((_H,), jnp.float32), pltpu.VMEM((16,), jnp.float32)]
            + [pltpu.SemaphoreType.DMA for _ in range(2 * nb)]
        ),
        compiler_params=pltpu.CompilerParams(use_tc_tiling_on_sc=False,
                                             needs_layout_passes=False),
    )
    def k(p_h, q_h, r_h, src3_h, dst3_h, w_h, b2_h, out_h, *scr):
        sidx_all, didx_all = scr[0], scr[1]
        prows = scr[2:2 + nb]
        qrows = scr[2 + nb:2 + 2 * nb]
        rrows = scr[2 + 2 * nb:2 + 3 * nb]
        obuf = scr[2 + 3 * nb:2 + 4 * nb]
        wvm = scr[2 + 4 * nb]
        b2vm = scr[3 + 4 * nb]
        gsem = scr[4 + 4 * nb:4 + 5 * nb]
        wsem = scr[4 + 5 * nb:4 + 6 * nb]
        c = lax.axis_index("c")
        s = lax.axis_index("s")
        wid = s * _NC + c
        base = wid * _EPT
        pltpu.sync_copy(src3_h.at[wid], sidx_all)
        pltpu.sync_copy(dst3_h.at[wid], didx_all)
        pltpu.sync_copy(w_h, wvm)
        pltpu.sync_copy(b2_h, b2vm)
        wvecs = [wvm[pl.ds(16 * t, 16)] for t in range(4)]
        b2v = b2vm[...]
        lane = lax.iota(jnp.int32, 16)
        last = jnp.full((16, 1), 15, jnp.int32)
        gdn = lax.GatherDimensionNumbers(
            offset_dims=(), collapsed_slice_dims=(0,), start_index_map=(0,))

        def stage_a(cc, j):
            off = pl.multiple_of(base + cc * _CH, 8)
            pltpu.make_async_copy(p_h.at[sidx_all.at[cc]], prows[j],
                                  gsem[j]).start()
            pltpu.make_async_copy(q_h.at[didx_all.at[cc]], qrows[j],
                                  gsem[j]).start()
            pltpu.make_async_copy(r_h.at[pl.ds(off, _CH)], rrows[j],
                                  gsem[j]).start()

        def stage_b(cc, j):
            off = pl.multiple_of(base + cc * _CH, 8)
            pltpu.make_async_copy(p_h.at[sidx_all.at[cc]], prows[j],
                                  gsem[j]).wait()
            pltpu.make_async_copy(q_h.at[didx_all.at[cc]], qrows[j],
                                  gsem[j]).wait()
            pltpu.make_async_copy(r_h.at[pl.ds(off, _CH)], rrows[j],
                                  gsem[j]).wait()

            def compute(ge, carry):
                ovec = jnp.zeros((16,), jnp.float32)
                for kk in range(16):
                    e = ge * 16 + kk
                    acc = jnp.zeros((16,), jnp.float32)
                    for g in range(2):
                        pv = prows[j][e, pl.ds(32 * g, 32)]
                        qv = qrows[j][e, pl.ds(32 * g, 32)]
                        rv = rrows[j][e, pl.ds(32 * g, 32)]
                        sv = jnp.maximum(pv + qv + rv, jnp.bfloat16(0))
                        bits = plsc.bitcast(sv, jnp.int32)
                        f_even = plsc.bitcast(
                            jax.lax.shift_left(bits, jnp.full((16,), 16, jnp.int32)),
                            jnp.float32)
                        f_odd = plsc.bitcast(
                            jnp.bitwise_and(bits, jnp.full((16,), -65536, jnp.int32)),
                            jnp.float32)
                        acc = acc + f_even * wvecs[2 * g] + f_odd * wvecs[2 * g + 1]
                    csum = plsc.cumsum(acc)
                    tot = lax.gather(
                        csum, last, gdn, (1,),
                        mode=lax.GatherScatterMode.PROMISE_IN_BOUNDS)
                    ovec = jnp.where(lane == kk, tot + b2v, ovec)
                obuf[j][ge, :] = ovec
                return carry

            lax.fori_loop(0, _CH // 16, compute, 0)
            orow = pl.multiple_of((base + cc * _CH) // 16, 8)
            pltpu.async_copy(obuf[j], out_h.at[pl.ds(orow, _CH // 16)], wsem[j])

        def wait_w(cc, j):
            orow = pl.multiple_of((base + cc * _CH) // 16, 8)
            pltpu.make_async_copy(obuf[j], out_h.at[pl.ds(orow, _CH // 16)],
                                  wsem[j]).wait()

        for j in range(nb):
            stage_a(j, j)
            if j > 0:
                stage_b(j - 1, j - 1)

        def body(g, carry):
            for j in range(nb):
                cc = g * nb + j
                wait_w(cc - nb, j)
                stage_a(cc, j)
                stage_b(cc - 1, (j - 1) % nb)
            return carry

        lax.fori_loop(1, _KCH // nb, body, 0)
        stage_b(_KCH - 1, nb - 1)
        for j in range(nb):
            wait_w(_KCH - nb + j, j)

    return k(p, q, r, src3, dst3, wm2_sc, bm2v)


def _tc_pre(x, w1lt, w1rt, b1l):
    blk = 1000

    def body(x_r, wl_r, wr_r, b_r, y1p_r, z1_r):
        xb = x_r[...]
        y = jnp.dot(xb, wl_r[...], preferred_element_type=jnp.float32)
        y1p_r[...] = jnp.concatenate(
            [y, jnp.ones((blk, 1), jnp.float32),
             jnp.zeros((blk, _W1 - _H - 1), jnp.float32)], axis=1).astype(jnp.bfloat16)
        z1_r[...] = jnp.dot(xb, wr_r[...], preferred_element_type=jnp.float32) + b_r[...]

    return pl.pallas_call(
        body,
        grid=(_N // blk,),
        in_specs=[
            pl.BlockSpec((blk, _D), lambda i: (i, 0)),
            pl.BlockSpec((_D, _H), lambda i: (0, 0)),
            pl.BlockSpec((_D, _H), lambda i: (0, 0)),
            pl.BlockSpec((1, _H), lambda i: (0, 0)),
        ],
        out_specs=[
            pl.BlockSpec((blk, _W1), lambda i: (i, 0)),
            pl.BlockSpec((blk, _H), lambda i: (i, 0)),
        ],
        out_shape=[
            jax.ShapeDtypeStruct((_N, _W1), jnp.bfloat16),
            jax.ShapeDtypeStruct((_N, _H), jnp.float32),
        ],
    )(x, w1lt, w1rt, b1l)


def _tc_mid(sa, sb, z1, w2lt, w2rt, b2l):
    blk = 1000

    def body(sa_r, sb_r, z_r, wl_r, wr_r, b_r, y2_r, z2_r, rc_r):
        ss = sa_r[...].astype(jnp.float32) + sb_r[...].astype(jnp.float32)
        rcnt = 1.0 / jnp.maximum(ss[:, _H:_H + 1], 1.0)
        h1 = jnp.maximum(ss[:, :_H] * rcnt + z_r[...], 0.0)
        y2_r[...] = jnp.dot(h1, wl_r[...], preferred_element_type=jnp.float32).astype(jnp.bfloat16)
        z2_r[...] = jnp.dot(h1, wr_r[...], preferred_element_type=jnp.float32) + b_r[...]
        rc_r[...] = rcnt

    return pl.pallas_call(
        body,
        grid=(_N // blk,),
        in_specs=[
            pl.BlockSpec((blk, _W1), lambda i: (i, 0)),
            pl.BlockSpec((blk, _W1), lambda i: (i, 0)),
            pl.BlockSpec((blk, _H), lambda i: (i, 0)),
            pl.BlockSpec((_H, _H), lambda i: (0, 0)),
            pl.BlockSpec((_H, _H), lambda i: (0, 0)),
            pl.BlockSpec((1, _H), lambda i: (0, 0)),
        ],
        out_specs=[
            pl.BlockSpec((blk, _H), lambda i: (i, 0)),
            pl.BlockSpec((blk, _H), lambda i: (i, 0)),
            pl.BlockSpec((blk, 1), lambda i: (i, 0)),
        ],
        out_shape=[
            jax.ShapeDtypeStruct((_N, _H), jnp.bfloat16),
            jax.ShapeDtypeStruct((_N, _H), jnp.float32),
            jax.ShapeDtypeStruct((_N, 1), jnp.float32),
        ],
    )(sa, sb, z1, w2lt, w2rt, b2l)


def _tc_post(sa, sb, z2, rcnt, at, bt):
    blk = 1000

    def body(sa_r, sb_r, z_r, rc_r, a_r, b_r, p_r, q_r):
        h2 = jnp.maximum(
            (sa_r[...].astype(jnp.float32) + sb_r[...].astype(jnp.float32))
            * rc_r[...] + z_r[...], 0.0)
        p_r[...] = jnp.dot(h2, a_r[...], preferred_element_type=jnp.float32).astype(jnp.bfloat16)
        q_r[...] = jnp.dot(h2, b_r[...], preferred_element_type=jnp.float32).astype(jnp.bfloat16)

    return pl.pallas_call(
        body,
        grid=(_N // blk,),
        in_specs=[
            pl.BlockSpec((blk, _H), lambda i: (i, 0)),
            pl.BlockSpec((blk, _H), lambda i: (i, 0)),
            pl.BlockSpec((blk, _H), lambda i: (i, 0)),
            pl.BlockSpec((blk, 1), lambda i: (i, 0)),
            pl.BlockSpec((_H, _H), lambda i: (0, 0)),
            pl.BlockSpec((_H, _H), lambda i: (0, 0)),
        ],
        out_specs=[
            pl.BlockSpec((blk, _H), lambda i: (i, 0)),
            pl.BlockSpec((blk, _H), lambda i: (i, 0)),
        ],
        out_shape=[
            jax.ShapeDtypeStruct((_N, _H), jnp.bfloat16),
            jax.ShapeDtypeStruct((_N, _H), jnp.bfloat16),
        ],
    )(sa, sb, z2, rcnt, at, bt)


def _tc_r(attr, ct, bm1):
    blk = 2048

    def body(a_r, c_r, b_r, r_r):
        r_r[...] = (jnp.dot(a_r[...], c_r[...], preferred_element_type=jnp.float32)
                    + b_r[...]).astype(jnp.bfloat16)

    return pl.pallas_call(
        body,
        grid=(_NP // blk,),
        in_specs=[
            pl.BlockSpec((blk, 16), lambda i: (i, 0)),
            pl.BlockSpec((16, _H), lambda i: (0, 0)),
            pl.BlockSpec((1, _H), lambda i: (0, 0)),
        ],
        out_specs=pl.BlockSpec((blk, _H), lambda i: (i, 0)),
        out_shape=jax.ShapeDtypeStruct((_NP, _H), jnp.bfloat16),
    )(attr, ct, bm1)


def kernel(x, mp_edge_index, pred_edge_index, pred_edge_attr,
           W1l, b1l, W1r, W2l, b2l, W2r, Wm1, bm1, Wm2, bm2):
    f32 = jnp.float32
    pad_e = _NP - _E
    mp_src = jnp.pad(mp_edge_index[0], (0, pad_e)).reshape(_NW, _KCH, _CH)
    mp_dst = jnp.pad(mp_edge_index[1], (0, pad_e),
                     constant_values=_N).reshape(_NW, _KCH, _CH)
    pr_src = jnp.pad(pred_edge_index[0], (0, pad_e)).reshape(_NW, _KCH, _CH)
    pr_dst = jnp.pad(pred_edge_index[1], (0, pad_e)).reshape(_NW, _KCH, _CH)
    attr_p = jnp.pad(pred_edge_attr, ((0, pad_e), (0, 0)))
    zeros96 = jnp.zeros((_ACC, _W1), jnp.bfloat16)
    zeros64 = jnp.zeros((_ACC, _H), jnp.bfloat16)

    y1p, z1 = _tc_pre(x, W1l.T, W1r.T, b1l.reshape(1, _H))
    s1 = _seg_sum(y1p, mp_src, mp_dst, zeros96, _W1)
    y2, z2, rcnt = _tc_mid(s1[0, :_N], s1[1, :_N], z1,
                           W2l.T, W2r.T, b2l.reshape(1, _H))
    s2 = _seg_sum(y2, mp_src, mp_dst, zeros64, _H)
    p, q = _tc_post(s2[0, :_N], s2[1, :_N], z2, rcnt,
                    Wm1[:, :_H].T, Wm1[:, _H:2 * _H].T)
    r = _tc_r(attr_p, Wm1[:, 2 * _H:].T, bm1.reshape(1, _H))
    w = Wm2.reshape(_H)
    wm2_sc = jnp.concatenate([w[0:32:2], w[1:32:2], w[32:64:2], w[33:64:2]])
    bm2v = jnp.full((16,), bm2[0], f32)
    o = _edge_mlp(p, q, r, pr_src, pr_dst, wm2_sc, bm2v)
    return o.reshape(_NP)[:_E]


# speedup vs baseline: 5.9067x; 1.1391x over previous
import functools

import jax
import jax.numpy as jnp
from jax import lax
from jax.experimental import pallas as pl
from jax.experimental.pallas import tpu as pltpu
from jax.experimental.pallas import tpu_sc as plsc

_N = 10000
_E = 320000
_D = 128
_H = 64

_NC = 2
_NS = 16
_NW = _NC * _NS
_CH = 128
_KCH = 80
_EPT = _CH * _KCH
_NP = _NW * _EPT
_W1 = 96
_ACC = 10240
_RPS = _ACC // _NS

_MESH = plsc.VectorSubcoreMesh(
    core_axis_name="c", subcore_axis_name="s", num_cores=_NC, num_subcores=_NS
)


def _seg_sum(table, src3, dst3, zeros, width):
    nb = 4

    @functools.partial(
        pl.kernel,
        out_type=jax.ShapeDtypeStruct((_NC, _ACC, width), jnp.bfloat16),
        mesh=_MESH,
        scratch_types=(
            [pltpu.VMEM((_KCH, _CH), jnp.int32) for _ in range(2)]
            + [pltpu.VMEM((_CH, width), jnp.bfloat16) for _ in range(nb)]
            + [pltpu.VMEM_SHARED((_ACC, width), jnp.bfloat16)]
            + [pltpu.SemaphoreType.DMA for _ in range(2 * nb)]
        ),
        compiler_params=pltpu.CompilerParams(use_tc_tiling_on_sc=False,
                                             needs_layout_passes=False),
    )
    def k(table_h, src3_h, dst3_h, zeros_h, out_h, *scr):
        sidx_all, didx_all = scr[0], scr[1]
        rows = scr[2:2 + nb]
        acc = scr[2 + nb]
        gsem = scr[3 + nb:3 + 2 * nb]
        asem = scr[3 + 2 * nb:3 + 3 * nb]
        c = lax.axis_index("c")
        s = lax.axis_index("s")
        wid = s * _NC + c
        pltpu.sync_copy(src3_h.at[wid], sidx_all)
        pltpu.sync_copy(dst3_h.at[wid], didx_all)
        pltpu.sync_copy(zeros_h.at[pl.ds(s * _RPS, _RPS)],
                        acc.at[pl.ds(s * _RPS, _RPS)])
        plsc.subcore_barrier()

        def ga(cc, j):
            pltpu.make_async_copy(table_h.at[sidx_all.at[cc]], rows[j],
                                  gsem[j]).start()

        def wg_sa(cc, j):
            pltpu.make_async_copy(table_h.at[sidx_all.at[cc]], rows[j],
                                  gsem[j]).wait()
            pltpu.make_async_copy(rows[j], acc.at[didx_all.at[cc]],
                                  asem[j]).start(add=True)

        def wa(cc, j):
            pltpu.make_async_copy(rows[j], acc.at[didx_all.at[cc]],
                                  asem[j]).wait()

        for j in range(nb):
            ga(j, j)
            if j > 0:
                wg_sa(j - 1, j - 1)

        def body(g, carry):
            for j in range(nb):
                cc = g * nb + j
                wa(cc - nb, j)
                ga(cc, j)
                wg_sa(cc - 1, (j - 1) % nb)
            return carry

        lax.fori_loop(1, _KCH // nb, body, 0)
        wg_sa(_KCH - 1, nb - 1)
        for j in range(nb):
            wa(_KCH - nb + j, j)
        plsc.subcore_barrier()
        pltpu.sync_copy(acc.at[pl.ds(s * _RPS, _RPS)],
                        out_h.at[c].at[pl.ds(s * _RPS, _RPS)])

    return k(table, src3, dst3, zeros)


def _edge_mlp(p, q, r, src3, dst3, wm2_sc, bm2v):
    nb = 4

    @functools.partial(
        pl.kernel,
        out_type=jax.ShapeDtypeStruct((_NP // 16, 16), jnp.float32),
        mesh=_MESH,
        scratch_types=(
            [pltpu.VMEM((_KCH, _CH), jnp.int32) for _ in range(2)]
            + [pltpu.VMEM((_CH, _H), jnp.bfloat16) for _ in range(2 * nb)]
            + [pltpu.VMEM((_CH * _H,), jnp.bfloat16) for _ in range(nb)]
            + [pltpu.VMEM((_CH // 16, 16), jnp.float32) for _ in range(nb)]
            + [pltpu.VMEM((_H,), jnp.float32), pltpu.VMEM((16,), jnp.float32)]
            + [pltpu.SemaphoreType.DMA for _ in range(2 * nb)]
        ),
        compiler_params=pltpu.CompilerParams(use_tc_tiling_on_sc=False,
                                             needs_layout_passes=False),
    )
    def k(p_h, q_h, r_h, src3_h, dst3_h, w_h, b2_h, out_h, *scr):
        sidx_all, didx_all = scr[0], scr[1]
        prows = scr[2:2 + nb]
        qrows = scr[2 + nb:2 + 2 * nb]
        rrows = scr[2 + 2 * nb:2 + 3 * nb]
        obuf = scr[2 + 3 * nb:2 + 4 * nb]
        wvm = scr[2 + 4 * nb]
        b2vm = scr[3 + 4 * nb]
        gsem = scr[4 + 4 * nb:4 + 5 * nb]
        wsem = scr[4 + 5 * nb:4 + 6 * nb]
        c = lax.axis_index("c")
        s = lax.axis_index("s")
        wid = s * _NC + c
        base = wid * _EPT
        pltpu.sync_copy(src3_h.at[wid], sidx_all)
        pltpu.sync_copy(dst3_h.at[wid], didx_all)
        pltpu.sync_copy(w_h, wvm)
        pltpu.sync_copy(b2_h, b2vm)
        wvecs = [wvm[pl.ds(16 * t, 16)] for t in range(4)]
        b2v = b2vm[...]
        lane = lax.iota(jnp.int32, 16)
        last = jnp.full((16, 1), 15, jnp.int32)
        gdn = lax.GatherDimensionNumbers(
            offset_dims=(), collapsed_slice_dims=(0,), start_index_map=(0,))

        def stage_a(cc, j):
            off = pl.multiple_of(base + cc * _CH, 8)
            pltpu.make_async_copy(p_h.at[sidx_all.at[cc]], prows[j],
                                  gsem[j]).start()
            pltpu.make_async_copy(q_h.at[didx_all.at[cc]], qrows[j],
                                  gsem[j]).start()
            off64 = pl.multiple_of((base + cc * _CH) * _H, 8)
            pltpu.make_async_copy(r_h.at[pl.ds(off64, _CH * _H)], rrows[j],
                                  gsem[j]).start()

        def stage_b(cc, j):
            off = pl.multiple_of(base + cc * _CH, 8)
            pltpu.make_async_copy(p_h.at[sidx_all.at[cc]], prows[j],
                                  gsem[j]).wait()
            pltpu.make_async_copy(q_h.at[didx_all.at[cc]], qrows[j],
                                  gsem[j]).wait()
            off64 = pl.multiple_of((base + cc * _CH) * _H, 8)
            pltpu.make_async_copy(r_h.at[pl.ds(off64, _CH * _H)], rrows[j],
                                  gsem[j]).wait()

            def compute(ge, carry):
                ovec = jnp.zeros((16,), jnp.float32)
                for kk in range(16):
                    e = ge * 16 + kk
                    acc = jnp.zeros((16,), jnp.float32)
                    for g in range(2):
                        pv = prows[j][e, pl.ds(32 * g, 32)]
                        qv = qrows[j][e, pl.ds(32 * g, 32)]
                        rv = rrows[j][pl.ds(e * _H + 32 * g, 32)]
                        sv = jnp.maximum(pv + qv + rv, jnp.bfloat16(0))
                        bits = plsc.bitcast(sv, jnp.int32)
                        f_even = plsc.bitcast(
                            jax.lax.shift_left(bits, jnp.full((16,), 16, jnp.int32)),
                            jnp.float32)
                        f_odd = plsc.bitcast(
                            jnp.bitwise_and(bits, jnp.full((16,), -65536, jnp.int32)),
                            jnp.float32)
                        acc = acc + f_even * wvecs[2 * g] + f_odd * wvecs[2 * g + 1]
                    csum = plsc.cumsum(acc)
                    tot = lax.gather(
                        csum, last, gdn, (1,),
                        mode=lax.GatherScatterMode.PROMISE_IN_BOUNDS)
                    ovec = jnp.where(lane == kk, tot + b2v, ovec)
                obuf[j][ge, :] = ovec
                return carry

            lax.fori_loop(0, _CH // 16, compute, 0)
            orow = pl.multiple_of((base + cc * _CH) // 16, 8)
            pltpu.async_copy(obuf[j], out_h.at[pl.ds(orow, _CH // 16)], wsem[j])

        def wait_w(cc, j):
            orow = pl.multiple_of((base + cc * _CH) // 16, 8)
            pltpu.make_async_copy(obuf[j], out_h.at[pl.ds(orow, _CH // 16)],
                                  wsem[j]).wait()

        for j in range(nb):
            stage_a(j, j)
            if j > 0:
                stage_b(j - 1, j - 1)

        def body(g, carry):
            for j in range(nb):
                cc = g * nb + j
                wait_w(cc - nb, j)
                stage_a(cc, j)
                stage_b(cc - 1, (j - 1) % nb)
            return carry

        lax.fori_loop(1, _KCH // nb, body, 0)
        stage_b(_KCH - 1, nb - 1)
        for j in range(nb):
            wait_w(_KCH - nb + j, j)

    return k(p, q, r, src3, dst3, wm2_sc, bm2v)


def _tc_pre(x, w1lt, w1rt, b1l):
    blk = 1000

    def body(x_r, wl_r, wr_r, b_r, y1p_r, z1_r):
        xb = x_r[...]
        y = jnp.dot(xb, wl_r[...], preferred_element_type=jnp.float32)
        y1p_r[...] = jnp.concatenate(
            [y, jnp.ones((blk, 1), jnp.float32),
             jnp.zeros((blk, _W1 - _H - 1), jnp.float32)], axis=1).astype(jnp.bfloat16)
        z1_r[...] = jnp.dot(xb, wr_r[...], preferred_element_type=jnp.float32) + b_r[...]

    return pl.pallas_call(
        body,
        grid=(_N // blk,),
        in_specs=[
            pl.BlockSpec((blk, _D), lambda i: (i, 0)),
            pl.BlockSpec((_D, _H), lambda i: (0, 0)),
            pl.BlockSpec((_D, _H), lambda i: (0, 0)),
            pl.BlockSpec((1, _H), lambda i: (0, 0)),
        ],
        out_specs=[
            pl.BlockSpec((blk, _W1), lambda i: (i, 0)),
            pl.BlockSpec((blk, _H), lambda i: (i, 0)),
        ],
        out_shape=[
            jax.ShapeDtypeStruct((_N, _W1), jnp.bfloat16),
            jax.ShapeDtypeStruct((_N, _H), jnp.float32),
        ],
    )(x, w1lt, w1rt, b1l)


def _tc_mid(sa, sb, z1, w2lt, w2rt, b2l):
    blk = 1000

    def body(sa_r, sb_r, z_r, wl_r, wr_r, b_r, y2_r, z2_r, rc_r):
        ss = sa_r[...].astype(jnp.float32) + sb_r[...].astype(jnp.float32)
        rcnt = 1.0 / jnp.maximum(ss[:, _H:_H + 1], 1.0)
        h1 = jnp.maximum(ss[:, :_H] * rcnt + z_r[...], 0.0)
        y2_r[...] = jnp.dot(h1, wl_r[...], preferred_element_type=jnp.float32).astype(jnp.bfloat16)
        z2_r[...] = jnp.dot(h1, wr_r[...], preferred_element_type=jnp.float32) + b_r[...]
        rc_r[...] = rcnt

    return pl.pallas_call(
        body,
        grid=(_N // blk,),
        in_specs=[
            pl.BlockSpec((blk, _W1), lambda i: (i, 0)),
            pl.BlockSpec((blk, _W1), lambda i: (i, 0)),
            pl.BlockSpec((blk, _H), lambda i: (i, 0)),
            pl.BlockSpec((_H, _H), lambda i: (0, 0)),
            pl.BlockSpec((_H, _H), lambda i: (0, 0)),
            pl.BlockSpec((1, _H), lambda i: (0, 0)),
        ],
        out_specs=[
            pl.BlockSpec((blk, _H), lambda i: (i, 0)),
            pl.BlockSpec((blk, _H), lambda i: (i, 0)),
            pl.BlockSpec((blk, 1), lambda i: (i, 0)),
        ],
        out_shape=[
            jax.ShapeDtypeStruct((_N, _H), jnp.bfloat16),
            jax.ShapeDtypeStruct((_N, _H), jnp.float32),
            jax.ShapeDtypeStruct((_N, 1), jnp.float32),
        ],
    )(sa, sb, z1, w2lt, w2rt, b2l)


def _tc_post(sa, sb, z2, rcnt, at, bt):
    blk = 1000

    def body(sa_r, sb_r, z_r, rc_r, a_r, b_r, p_r, q_r):
        h2 = jnp.maximum(
            (sa_r[...].astype(jnp.float32) + sb_r[...].astype(jnp.float32))
            * rc_r[...] + z_r[...], 0.0)
        p_r[...] = jnp.dot(h2, a_r[...], preferred_element_type=jnp.float32).astype(jnp.bfloat16)
        q_r[...] = jnp.dot(h2, b_r[...], preferred_element_type=jnp.float32).astype(jnp.bfloat16)

    return pl.pallas_call(
        body,
        grid=(_N // blk,),
        in_specs=[
            pl.BlockSpec((blk, _H), lambda i: (i, 0)),
            pl.BlockSpec((blk, _H), lambda i: (i, 0)),
            pl.BlockSpec((blk, _H), lambda i: (i, 0)),
            pl.BlockSpec((blk, 1), lambda i: (i, 0)),
            pl.BlockSpec((_H, _H), lambda i: (0, 0)),
            pl.BlockSpec((_H, _H), lambda i: (0, 0)),
        ],
        out_specs=[
            pl.BlockSpec((blk, _H), lambda i: (i, 0)),
            pl.BlockSpec((blk, _H), lambda i: (i, 0)),
        ],
        out_shape=[
            jax.ShapeDtypeStruct((_N, _H), jnp.bfloat16),
            jax.ShapeDtypeStruct((_N, _H), jnp.bfloat16),
        ],
    )(sa, sb, z2, rcnt, at, bt)


def _tc_r8(attr8, ct8, bm18):
    blk = 400

    def body(a_r, c_r, b_r, o_r):
        rr = (jnp.dot(a_r[...], c_r[...], preferred_element_type=jnp.float32)
              + b_r[...]).astype(jnp.bfloat16)
        o_r[...] = rr.reshape(blk * 512)

    return pl.pallas_call(
        body,
        grid=(_E // 8 // blk,),
        in_specs=[
            pl.BlockSpec((blk, 128), lambda i: (i, 0)),
            pl.BlockSpec((128, 512), lambda i: (0, 0)),
            pl.BlockSpec((1, 512), lambda i: (0, 0)),
        ],
        out_specs=pl.BlockSpec((blk * 512,), lambda i: (i,)),
        out_shape=jax.ShapeDtypeStruct((_NP * _H,), jnp.bfloat16),
    )(attr8, ct8, bm18)


def kernel(x, mp_edge_index, pred_edge_index, pred_edge_attr,
           W1l, b1l, W1r, W2l, b2l, W2r, Wm1, bm1, Wm2, bm2):
    f32 = jnp.float32
    pad_e = _NP - _E
    mp_src = jnp.pad(mp_edge_index[0], (0, pad_e)).reshape(_NW, _KCH, _CH)
    mp_dst = jnp.pad(mp_edge_index[1], (0, pad_e),
                     constant_values=_N).reshape(_NW, _KCH, _CH)
    pr_src = jnp.pad(pred_edge_index[0], (0, pad_e)).reshape(_NW, _KCH, _CH)
    pr_dst = jnp.pad(pred_edge_index[1], (0, pad_e)).reshape(_NW, _KCH, _CH)
    zeros96 = jnp.zeros((_ACC, _W1), jnp.bfloat16)
    zeros64 = jnp.zeros((_ACC, _H), jnp.bfloat16)

    y1p, z1 = _tc_pre(x, W1l.T, W1r.T, b1l.reshape(1, _H))
    s1 = _seg_sum(y1p, mp_src, mp_dst, zeros96, _W1)
    y2, z2, rcnt = _tc_mid(s1[0, :_N], s1[1, :_N], z1,
                           W2l.T, W2r.T, b2l.reshape(1, _H))
    s2 = _seg_sum(y2, mp_src, mp_dst, zeros64, _H)
    p, q = _tc_post(s2[0, :_N], s2[1, :_N], z2, rcnt,
                    Wm1[:, :_H].T, Wm1[:, _H:2 * _H].T)
    attr8 = pred_edge_attr.reshape(_E // 8, 128)
    ct8 = jnp.kron(jnp.eye(8, dtype=f32), Wm1[:, 2 * _H:].T)
    bm18 = jnp.tile(bm1, 8).reshape(1, 512)
    r = _tc_r8(attr8, ct8, bm18)
    w = Wm2.reshape(_H)
    wm2_sc = jnp.concatenate([w[0:32:2], w[1:32:2], w[32:64:2], w[33:64:2]])
    bm2v = jnp.full((16,), bm2[0], f32)
    o = _edge_mlp(p, q, r, pr_src, pr_dst, wm2_sc, bm2v)
    return o.reshape(_NP)[:_E]
